# baseline (speedup 1.0000x reference)
import jax
import jax.numpy as jnp
from jax import lax
from jax.experimental import pallas as pl
from jax.experimental.pallas import tpu as pltpu

_B, _S, _D, _E = 4, 8192, 768, 64
_CHUNK = 512
_NC = _S // _CHUNK


def _gate_tail(m, wh, bh, wo, bo):
    h = jnp.dot(m, wh, preferred_element_type=jnp.float32) + bh
    h = h * jax.nn.sigmoid(h)
    logits = jnp.dot(h, wo, preferred_element_type=jnp.float32) + bo
    iota = lax.broadcasted_iota(jnp.int32, logits.shape, 1)
    v1 = jnp.max(logits, axis=1, keepdims=True)
    i1 = jnp.min(jnp.where(logits == v1, iota, _E), axis=1, keepdims=True)
    masked = jnp.where(iota == i1, -jnp.inf, logits)
    v2 = jnp.max(masked, axis=1, keepdims=True)
    i2 = jnp.min(jnp.where(masked == v2, iota, _E), axis=1, keepdims=True)
    e2 = jnp.exp(v2 - v1)
    denom = 1.0 + e2
    w = jnp.concatenate([1.0 / denom, e2 / denom], axis=1)
    i = jnp.concatenate([i1, i2], axis=1)
    return w, i


def _tc_body(x_ref, wh_ref, bh_ref, wo_ref, bo_ref, w_ref, i_ref, acc_ref):
    b = pl.program_id(0)
    c = pl.program_id(1)
    partial = jnp.sum(x_ref[0], axis=0, keepdims=True)

    @pl.when(c == 0)
    def _():
        acc_ref[pl.ds(b, 1), :] = partial

    @pl.when(c > 0)
    def _():
        acc_ref[pl.ds(b, 1), :] += partial

    @pl.when((b == _B - 1) & (c == _NC - 1))
    def _():
        m = acc_ref[...] * (1.0 / _S)
        w, i = _gate_tail(m, wh_ref[...], bh_ref[...], wo_ref[...], bo_ref[...])
        w_ref[...] = w
        i_ref[...] = i


def kernel(x, W_hidden, b_hidden, W_out, b_out):
    bh = b_hidden.reshape(1, _D)
    bo = b_out.reshape(1, _E)
    w, i = pl.pallas_call(
        _tc_body,
        grid=(_B, _NC),
        in_specs=[
            pl.BlockSpec((1, _CHUNK, _D), lambda b, c: (b, c, 0)),
            pl.BlockSpec((_D, _D), lambda b, c: (0, 0)),
            pl.BlockSpec((1, _D), lambda b, c: (0, 0)),
            pl.BlockSpec((_D, _E), lambda b, c: (0, 0)),
            pl.BlockSpec((1, _E), lambda b, c: (0, 0)),
        ],
        out_specs=[
            pl.BlockSpec((_B, 2), lambda b, c: (0, 0)),
            pl.BlockSpec((_B, 2), lambda b, c: (0, 0)),
        ],
        out_shape=[
            jax.ShapeDtypeStruct((_B, 2), jnp.float32),
            jax.ShapeDtypeStruct((_B, 2), jnp.int32),
        ],
        scratch_shapes=[pltpu.VMEM((_B, _D), jnp.float32)],
    )(x, W_hidden, bh, W_out, bo)
    return w, i


# speedup vs baseline: 1.5682x; 1.5682x over previous
import jax
import jax.numpy as jnp
from jax import lax
from jax.experimental import pallas as pl
from jax.experimental.pallas import tpu as pltpu

_B, _S, _D, _E = 4, 8192, 768, 64
_CHUNK = 1024
_NC = _S // _CHUNK


def _gate_tail(m, wh, bh, wo, bo):
    h = jnp.dot(m, wh, preferred_element_type=jnp.float32) + bh
    h = h * jax.nn.sigmoid(h)
    logits = jnp.dot(h, wo, preferred_element_type=jnp.float32) + bo
    iota = lax.broadcasted_iota(jnp.int32, logits.shape, 1)
    v1 = jnp.max(logits, axis=1, keepdims=True)
    i1 = jnp.min(jnp.where(logits == v1, iota, _E), axis=1, keepdims=True)
    masked = jnp.where(iota == i1, -jnp.inf, logits)
    v2 = jnp.max(masked, axis=1, keepdims=True)
    i2 = jnp.min(jnp.where(masked == v2, iota, _E), axis=1, keepdims=True)
    e2 = jnp.exp(v2 - v1)
    denom = 1.0 + e2
    w = jnp.concatenate([1.0 / denom, e2 / denom], axis=1)
    i = jnp.concatenate([i1, i2], axis=1)
    return w, i


def _tc_body(x_ref, wh_ref, bh_ref, wo_ref, bo_ref, w_ref, i_ref, acc_ref):
    c = pl.program_id(0)
    partial = jnp.sum(x_ref[...], axis=1)

    @pl.when(c == 0)
    def _():
        acc_ref[...] = partial

    @pl.when(c > 0)
    def _():
        acc_ref[...] += partial

    @pl.when(c == _NC - 1)
    def _():
        m = acc_ref[...] * (1.0 / _S)
        w, i = _gate_tail(m, wh_ref[...], bh_ref[...], wo_ref[...], bo_ref[...])
        w_ref[...] = w
        i_ref[...] = i


def kernel(x, W_hidden, b_hidden, W_out, b_out):
    bh = b_hidden.reshape(1, _D)
    bo = b_out.reshape(1, _E)
    w, i = pl.pallas_call(
        _tc_body,
        grid=(_NC,),
        in_specs=[
            pl.BlockSpec((_B, _CHUNK, _D), lambda c: (0, c, 0)),
            pl.BlockSpec((_D, _D), lambda c: (0, 0)),
            pl.BlockSpec((1, _D), lambda c: (0, 0)),
            pl.BlockSpec((_D, _E), lambda c: (0, 0)),
            pl.BlockSpec((1, _E), lambda c: (0, 0)),
        ],
        out_specs=[
            pl.BlockSpec((_B, 2), lambda c: (0, 0)),
            pl.BlockSpec((_B, 2), lambda c: (0, 0)),
        ],
        out_shape=[
            jax.ShapeDtypeStruct((_B, 2), jnp.float32),
            jax.ShapeDtypeStruct((_B, 2), jnp.int32),
        ],
        scratch_shapes=[pltpu.VMEM((_B, _D), jnp.float32)],
    )(x, W_hidden, bh, W_out, bo)
    return w, i


# speedup vs baseline: 1.5921x; 1.0153x over previous
import jax
import jax.numpy as jnp
from jax import lax
from jax.experimental import pallas as pl
from jax.experimental.pallas import tpu as pltpu

_B, _S, _D, _E = 4, 8192, 768, 64
_CHUNK = 2048
_NC = _S // _CHUNK


def _gate_tail(m, wh, bh, wo, bo):
    h = jnp.dot(m, wh, preferred_element_type=jnp.float32) + bh
    h = h * jax.nn.sigmoid(h)
    logits = jnp.dot(h, wo, preferred_element_type=jnp.float32) + bo
    iota = lax.broadcasted_iota(jnp.int32, logits.shape, 1)
    v1 = jnp.max(logits, axis=1, keepdims=True)
    i1 = jnp.min(jnp.where(logits == v1, iota, _E), axis=1, keepdims=True)
    masked = jnp.where(iota == i1, -jnp.inf, logits)
    v2 = jnp.max(masked, axis=1, keepdims=True)
    i2 = jnp.min(jnp.where(masked == v2, iota, _E), axis=1, keepdims=True)
    e2 = jnp.exp(v2 - v1)
    denom = 1.0 + e2
    w = jnp.concatenate([1.0 / denom, e2 / denom], axis=1)
    i = jnp.concatenate([i1, i2], axis=1)
    return w, i


def _tc_body(x_ref, wh_ref, bh_ref, wo_ref, bo_ref, w_ref, i_ref, acc_ref):
    b = pl.program_id(0)
    c = pl.program_id(1)
    partial = jnp.sum(x_ref[0], axis=0, keepdims=True)

    @pl.when(c == 0)
    def _():
        acc_ref[pl.ds(b, 1), :] = partial

    @pl.when(c > 0)
    def _():
        acc_ref[pl.ds(b, 1), :] += partial

    @pl.when((b == _B - 1) & (c == _NC - 1))
    def _():
        m = acc_ref[...] * (1.0 / _S)
        w, i = _gate_tail(m, wh_ref[...], bh_ref[...], wo_ref[...], bo_ref[...])
        w_ref[...] = w
        i_ref[...] = i


def kernel(x, W_hidden, b_hidden, W_out, b_out):
    bh = b_hidden.reshape(1, _D)
    bo = b_out.reshape(1, _E)
    w, i = pl.pallas_call(
        _tc_body,
        grid=(_B, _NC),
        in_specs=[
            pl.BlockSpec((1, _CHUNK, _D), lambda b, c: (b, c, 0)),
            pl.BlockSpec((_D, _D), lambda b, c: (0, 0)),
            pl.BlockSpec((1, _D), lambda b, c: (0, 0)),
            pl.BlockSpec((_D, _E), lambda b, c: (0, 0)),
            pl.BlockSpec((1, _E), lambda b, c: (0, 0)),
        ],
        out_specs=[
            pl.BlockSpec((_B, 2), lambda b, c: (0, 0)),
            pl.BlockSpec((_B, 2), lambda b, c: (0, 0)),
        ],
        out_shape=[
            jax.ShapeDtypeStruct((_B, 2), jnp.float32),
            jax.ShapeDtypeStruct((_B, 2), jnp.int32),
        ],
        scratch_shapes=[pltpu.VMEM((_B, _D), jnp.float32)],
    )(x, W_hidden, bh, W_out, bo)
    return w, i
